# baseline (device time: 3512092 ns/iter reference)
import jax
import jax.numpy as jnp
from jax import lax
from jax.experimental import pallas as pl
from jax.experimental.pallas import tpu as pltpu

W = 32
M = 8192
N = 4096
CH = M // W


def _ar_body(p_ref, o_ref, local_v, acc_v, recv_v, fwd_v,
             send_sem, recv_sem, local_sem, store_sem, credit_sem):
    my = lax.axis_index("i")
    left = lax.rem(my + W - 1, W)
    right = lax.rem(my + 1, W)

    barrier = pltpu.get_barrier_semaphore()
    for nbr in (left, right):
        pl.semaphore_signal(barrier, inc=1, device_id=(nbr,),
                            device_id_type=pl.DeviceIdType.MESH)
    pl.semaphore_wait(barrier, 2)

    def rows(c):
        return pl.ds(c * CH, CH)

    cp = pltpu.make_async_copy(p_ref.at[rows(my), :], acc_v, local_sem)
    cp.start()
    cp.wait()

    for s in range(W - 1):
        recv_chunk = lax.rem(my - s - 1 + 2 * W, W)
        lp = pltpu.make_async_copy(p_ref.at[rows(recv_chunk), :], local_v,
                                   local_sem)
        lp.start()
        if s > 0:
            pl.semaphore_wait(credit_sem, 1)
        rdma = pltpu.make_async_remote_copy(
            src_ref=acc_v, dst_ref=recv_v,
            send_sem=send_sem, recv_sem=recv_sem,
            device_id=(right,), device_id_type=pl.DeviceIdType.MESH,
        )
        rdma.start()
        rdma.wait()
        lp.wait()
        acc_v[...] = recv_v[...] + local_v[...]
        pl.semaphore_signal(credit_sem, inc=1, device_id=(left,),
                            device_id_type=pl.DeviceIdType.MESH)

    acc_v[...] = jnp.maximum(acc_v[...], 0.0)
    own = lax.rem(my + 1, W)
    st = pltpu.make_async_copy(acc_v, o_ref.at[rows(own), :], store_sem)
    st.start()
    st.wait()

    for t in range(W - 1):
        src = acc_v if t == 0 else fwd_v
        pl.semaphore_wait(credit_sem, 1)
        rdma = pltpu.make_async_remote_copy(
            src_ref=src, dst_ref=recv_v,
            send_sem=send_sem, recv_sem=recv_sem,
            device_id=(right,), device_id_type=pl.DeviceIdType.MESH,
        )
        rdma.start()
        rdma.wait()
        rchunk = lax.rem(my - t + 2 * W, W)
        st = pltpu.make_async_copy(recv_v, o_ref.at[rows(rchunk), :],
                                   store_sem)
        st.start()
        if t < W - 2:
            fwd_v[...] = recv_v[...]
        st.wait()
        if t < W - 2:
            pl.semaphore_signal(credit_sem, inc=1, device_id=(left,),
                                device_id_type=pl.DeviceIdType.MESH)


def _allreduce_relu(partial):
    return pl.pallas_call(
        _ar_body,
        out_shape=jax.ShapeDtypeStruct((M, N), jnp.float32),
        in_specs=[pl.BlockSpec(memory_space=pltpu.MemorySpace.HBM)],
        out_specs=pl.BlockSpec(memory_space=pltpu.MemorySpace.HBM),
        scratch_shapes=[
            pltpu.VMEM((CH, N), jnp.float32),
            pltpu.VMEM((CH, N), jnp.float32),
            pltpu.VMEM((CH, N), jnp.float32),
            pltpu.VMEM((CH, N), jnp.float32),
            pltpu.SemaphoreType.DMA,
            pltpu.SemaphoreType.DMA,
            pltpu.SemaphoreType.DMA,
            pltpu.SemaphoreType.DMA,
            pltpu.SemaphoreType.REGULAR,
        ],
        compiler_params=pltpu.CompilerParams(collective_id=0),
    )(partial)


def kernel(x, w_mat):
    partial = lax.dot_general(
        x, w_mat, (((1,), (0,)), ((), ())),
        precision=lax.Precision.HIGHEST,
        preferred_element_type=jnp.float32,
    )
    return _allreduce_relu(partial)


# device time: 3149282 ns/iter; 1.1152x vs baseline; 1.1152x over previous
import jax
import jax.numpy as jnp
from jax import lax
from jax.experimental import pallas as pl
from jax.experimental.pallas import tpu as pltpu

W = 32
M = 8192
N = 4096
CH = M // W
N2 = N // 2


def _ar_body(p_ref, o_ref,
             localR, accR, recvR, fwdR,
             localL, accL, recvL, fwdL,
             sendR_sem, recvR_sem, localR_sem, storeR_sem,
             sendL_sem, recvL_sem, localL_sem, storeL_sem,
             creditR, creditL):
    my = lax.axis_index("i")
    left = lax.rem(my + W - 1, W)
    right = lax.rem(my + 1, W)

    barrier = pltpu.get_barrier_semaphore()
    for nbr in (left, right):
        pl.semaphore_signal(barrier, inc=1, device_id=(nbr,),
                            device_id_type=pl.DeviceIdType.MESH)
    pl.semaphore_wait(barrier, 2)

    def rows(c):
        return pl.ds(c * CH, CH)

    colsR = pl.ds(0, N2)
    colsL = pl.ds(N2, N2)

    cpR = pltpu.make_async_copy(p_ref.at[rows(my), colsR], accR, localR_sem)
    cpL = pltpu.make_async_copy(p_ref.at[rows(my), colsL], accL, localL_sem)
    cpR.start()
    cpL.start()
    cpR.wait()
    cpL.wait()

    for s in range(W - 1):
        rR = lax.rem(my - s - 1 + 2 * W, W)
        rL = lax.rem(my + s + 1, W)
        lpR = pltpu.make_async_copy(p_ref.at[rows(rR), colsR], localR,
                                    localR_sem)
        lpL = pltpu.make_async_copy(p_ref.at[rows(rL), colsL], localL,
                                    localL_sem)
        lpR.start()
        lpL.start()
        if s > 0:
            pl.semaphore_wait(creditR, 1)
            pl.semaphore_wait(creditL, 1)
        rdmaR = pltpu.make_async_remote_copy(
            src_ref=accR, dst_ref=recvR,
            send_sem=sendR_sem, recv_sem=recvR_sem,
            device_id=(right,), device_id_type=pl.DeviceIdType.MESH,
        )
        rdmaL = pltpu.make_async_remote_copy(
            src_ref=accL, dst_ref=recvL,
            send_sem=sendL_sem, recv_sem=recvL_sem,
            device_id=(left,), device_id_type=pl.DeviceIdType.MESH,
        )
        rdmaR.start()
        rdmaL.start()
        rdmaR.wait()
        rdmaL.wait()
        lpR.wait()
        lpL.wait()
        accR[...] = recvR[...] + localR[...]
        accL[...] = recvL[...] + localL[...]
        pl.semaphore_signal(creditR, inc=1, device_id=(left,),
                            device_id_type=pl.DeviceIdType.MESH)
        pl.semaphore_signal(creditL, inc=1, device_id=(right,),
                            device_id_type=pl.DeviceIdType.MESH)

    accR[...] = jnp.maximum(accR[...], 0.0)
    accL[...] = jnp.maximum(accL[...], 0.0)
    stR = pltpu.make_async_copy(accR, o_ref.at[rows(lax.rem(my + 1, W)), colsR],
                                storeR_sem)
    stL = pltpu.make_async_copy(accL, o_ref.at[rows(left), colsL], storeL_sem)
    stR.start()
    stL.start()
    stR.wait()
    stL.wait()

    for t in range(W - 1):
        srcR = accR if t == 0 else fwdR
        srcL = accL if t == 0 else fwdL
        pl.semaphore_wait(creditR, 1)
        pl.semaphore_wait(creditL, 1)
        rdmaR = pltpu.make_async_remote_copy(
            src_ref=srcR, dst_ref=recvR,
            send_sem=sendR_sem, recv_sem=recvR_sem,
            device_id=(right,), device_id_type=pl.DeviceIdType.MESH,
        )
        rdmaL = pltpu.make_async_remote_copy(
            src_ref=srcL, dst_ref=recvL,
            send_sem=sendL_sem, recv_sem=recvL_sem,
            device_id=(left,), device_id_type=pl.DeviceIdType.MESH,
        )
        rdmaR.start()
        rdmaL.start()
        rdmaR.wait()
        rdmaL.wait()
        cR = lax.rem(my - t + 2 * W, W)
        cL = lax.rem(my + t, W)
        stR = pltpu.make_async_copy(recvR, o_ref.at[rows(cR), colsR],
                                    storeR_sem)
        stL = pltpu.make_async_copy(recvL, o_ref.at[rows(cL), colsL],
                                    storeL_sem)
        stR.start()
        stL.start()
        if t < W - 2:
            fwdR[...] = recvR[...]
            fwdL[...] = recvL[...]
        stR.wait()
        stL.wait()
        if t < W - 2:
            pl.semaphore_signal(creditR, inc=1, device_id=(left,),
                                device_id_type=pl.DeviceIdType.MESH)
            pl.semaphore_signal(creditL, inc=1, device_id=(right,),
                                device_id_type=pl.DeviceIdType.MESH)


def _allreduce_relu(partial):
    return pl.pallas_call(
        _ar_body,
        out_shape=jax.ShapeDtypeStruct((M, N), jnp.float32),
        in_specs=[pl.BlockSpec(memory_space=pltpu.MemorySpace.HBM)],
        out_specs=pl.BlockSpec(memory_space=pltpu.MemorySpace.HBM),
        scratch_shapes=[
            pltpu.VMEM((CH, N2), jnp.float32),
            pltpu.VMEM((CH, N2), jnp.float32),
            pltpu.VMEM((CH, N2), jnp.float32),
            pltpu.VMEM((CH, N2), jnp.float32),
            pltpu.VMEM((CH, N2), jnp.float32),
            pltpu.VMEM((CH, N2), jnp.float32),
            pltpu.VMEM((CH, N2), jnp.float32),
            pltpu.VMEM((CH, N2), jnp.float32),
            pltpu.SemaphoreType.DMA,
            pltpu.SemaphoreType.DMA,
            pltpu.SemaphoreType.DMA,
            pltpu.SemaphoreType.DMA,
            pltpu.SemaphoreType.DMA,
            pltpu.SemaphoreType.DMA,
            pltpu.SemaphoreType.DMA,
            pltpu.SemaphoreType.DMA,
            pltpu.SemaphoreType.REGULAR,
            pltpu.SemaphoreType.REGULAR,
        ],
        compiler_params=pltpu.CompilerParams(collective_id=0),
    )(partial)


def kernel(x, w_mat):
    partial = lax.dot_general(
        x, w_mat, (((1,), (0,)), ((), ())),
        preferred_element_type=jnp.float32,
    )
    return _allreduce_relu(partial)


# device time: 1769737 ns/iter; 1.9845x vs baseline; 1.7795x over previous
import jax
import jax.numpy as jnp
from jax import lax
from jax.experimental import pallas as pl
from jax.experimental.pallas import tpu as pltpu

W = 32
M = 8192
N = 4096
CH = M // W
N2 = N // 2


def _mesh_logical_order():
    coords = [(x, y, z) for x in range(2) for y in range(4) for z in range(4)]
    ring = []
    for z in range(4):
        for yi, y in enumerate(range(4)):
            row = sorted((c for c in coords if c[1] == y and c[2] == z),
                         reverse=bool(yi % 2))
            ring.extend(row)
    return ring


def _hamiltonian_cycle():
    p0 = []
    for z in range(4):
        ys = range(4) if z % 2 == 0 else range(3, -1, -1)
        p0.extend((0, y, z) for y in ys)
    return p0 + [(1, y, z) for (_, y, z) in reversed(p0)]


_LOGICAL = _mesh_logical_order()
_L_OF = {c: i for i, c in enumerate(_LOGICAL)}
_RING = [_L_OF[c] for c in _hamiltonian_cycle()]
POS = [0] * W
SUCC = [0] * W
PRED = [0] * W
for _p, _l in enumerate(_RING):
    POS[_l] = _p
    SUCC[_l] = _RING[(_p + 1) % W]
    PRED[_l] = _RING[(_p - 1) % W]


def _ar_body(meta_ref, p_ref, o_ref,
             localR, accR, recvR, fwdR,
             localL, accL, recvL, fwdL,
             sendR_sem, recvR_sem, localR_sem, storeR_sem,
             sendL_sem, recvL_sem, localL_sem, storeL_sem,
             creditR, creditL):
    pos = meta_ref[0]
    succ = meta_ref[1]
    pred = meta_ref[2]

    barrier = pltpu.get_barrier_semaphore()
    for nbr in (pred, succ):
        pl.semaphore_signal(barrier, inc=1, device_id=(nbr,),
                            device_id_type=pl.DeviceIdType.MESH)
    pl.semaphore_wait(barrier, 2)

    def rows(c):
        return pl.ds(c * CH, CH)

    colsR = pl.ds(0, N2)
    colsL = pl.ds(N2, N2)

    cpR = pltpu.make_async_copy(p_ref.at[rows(pos), colsR], accR, localR_sem)
    cpL = pltpu.make_async_copy(p_ref.at[rows(pos), colsL], accL, localL_sem)
    cpR.start()
    cpL.start()
    cpR.wait()
    cpL.wait()

    for s in range(W - 1):
        rR = lax.rem(pos - s - 1 + 2 * W, W)
        rL = lax.rem(pos + s + 1, W)
        lpR = pltpu.make_async_copy(p_ref.at[rows(rR), colsR], localR,
                                    localR_sem)
        lpL = pltpu.make_async_copy(p_ref.at[rows(rL), colsL], localL,
                                    localL_sem)
        lpR.start()
        lpL.start()
        if s > 0:
            pl.semaphore_wait(creditR, 1)
            pl.semaphore_wait(creditL, 1)
        rdmaR = pltpu.make_async_remote_copy(
            src_ref=accR, dst_ref=recvR,
            send_sem=sendR_sem, recv_sem=recvR_sem,
            device_id=(succ,), device_id_type=pl.DeviceIdType.MESH,
        )
        rdmaL = pltpu.make_async_remote_copy(
            src_ref=accL, dst_ref=recvL,
            send_sem=sendL_sem, recv_sem=recvL_sem,
            device_id=(pred,), device_id_type=pl.DeviceIdType.MESH,
        )
        rdmaR.start()
        rdmaL.start()
        rdmaR.wait()
        rdmaL.wait()
        lpR.wait()
        lpL.wait()
        accR[...] = recvR[...] + localR[...]
        accL[...] = recvL[...] + localL[...]
        pl.semaphore_signal(creditR, inc=1, device_id=(pred,),
                            device_id_type=pl.DeviceIdType.MESH)
        pl.semaphore_signal(creditL, inc=1, device_id=(succ,),
                            device_id_type=pl.DeviceIdType.MESH)

    accR[...] = jnp.maximum(accR[...], 0.0)
    accL[...] = jnp.maximum(accL[...], 0.0)
    stR = pltpu.make_async_copy(
        accR, o_ref.at[rows(lax.rem(pos + 1, W)), colsR], storeR_sem)
    stL = pltpu.make_async_copy(
        accL, o_ref.at[rows(lax.rem(pos + W - 1, W)), colsL], storeL_sem)
    stR.start()
    stL.start()
    stR.wait()
    stL.wait()

    for t in range(W - 1):
        srcR = accR if t == 0 else fwdR
        srcL = accL if t == 0 else fwdL
        pl.semaphore_wait(creditR, 1)
        pl.semaphore_wait(creditL, 1)
        rdmaR = pltpu.make_async_remote_copy(
            src_ref=srcR, dst_ref=recvR,
            send_sem=sendR_sem, recv_sem=recvR_sem,
            device_id=(succ,), device_id_type=pl.DeviceIdType.MESH,
        )
        rdmaL = pltpu.make_async_remote_copy(
            src_ref=srcL, dst_ref=recvL,
            send_sem=sendL_sem, recv_sem=recvL_sem,
            device_id=(pred,), device_id_type=pl.DeviceIdType.MESH,
        )
        rdmaR.start()
        rdmaL.start()
        rdmaR.wait()
        rdmaL.wait()
        cR = lax.rem(pos - t + 2 * W, W)
        cL = lax.rem(pos + t, W)
        stR = pltpu.make_async_copy(recvR, o_ref.at[rows(cR), colsR],
                                    storeR_sem)
        stL = pltpu.make_async_copy(recvL, o_ref.at[rows(cL), colsL],
                                    storeL_sem)
        stR.start()
        stL.start()
        if t < W - 2:
            fwdR[...] = recvR[...]
            fwdL[...] = recvL[...]
        stR.wait()
        stL.wait()
        if t < W - 2:
            pl.semaphore_signal(creditR, inc=1, device_id=(pred,),
                                device_id_type=pl.DeviceIdType.MESH)
            pl.semaphore_signal(creditL, inc=1, device_id=(succ,),
                                device_id_type=pl.DeviceIdType.MESH)


def _allreduce_relu(meta, partial):
    return pl.pallas_call(
        _ar_body,
        out_shape=jax.ShapeDtypeStruct((M, N), jnp.float32),
        in_specs=[
            pl.BlockSpec(memory_space=pltpu.MemorySpace.SMEM),
            pl.BlockSpec(memory_space=pltpu.MemorySpace.HBM),
        ],
        out_specs=pl.BlockSpec(memory_space=pltpu.MemorySpace.HBM),
        scratch_shapes=[
            pltpu.VMEM((CH, N2), jnp.float32),
            pltpu.VMEM((CH, N2), jnp.float32),
            pltpu.VMEM((CH, N2), jnp.float32),
            pltpu.VMEM((CH, N2), jnp.float32),
            pltpu.VMEM((CH, N2), jnp.float32),
            pltpu.VMEM((CH, N2), jnp.float32),
            pltpu.VMEM((CH, N2), jnp.float32),
            pltpu.VMEM((CH, N2), jnp.float32),
            pltpu.SemaphoreType.DMA,
            pltpu.SemaphoreType.DMA,
            pltpu.SemaphoreType.DMA,
            pltpu.SemaphoreType.DMA,
            pltpu.SemaphoreType.DMA,
            pltpu.SemaphoreType.DMA,
            pltpu.SemaphoreType.DMA,
            pltpu.SemaphoreType.DMA,
            pltpu.SemaphoreType.REGULAR,
            pltpu.SemaphoreType.REGULAR,
        ],
        compiler_params=pltpu.CompilerParams(collective_id=0),
    )(meta, partial)


def kernel(x, w_mat):
    partial = lax.dot_general(
        x, w_mat, (((1,), (0,)), ((), ())),
        preferred_element_type=jnp.float32,
    )
    my = lax.axis_index("i")
    meta = jnp.stack([
        jnp.array(POS, jnp.int32)[my],
        jnp.array(SUCC, jnp.int32)[my],
        jnp.array(PRED, jnp.int32)[my],
    ])
    return _allreduce_relu(meta, partial)


# device time: 1686010 ns/iter; 2.0831x vs baseline; 1.0497x over previous
import jax
import jax.numpy as jnp
from jax import lax
from jax.experimental import pallas as pl
from jax.experimental.pallas import tpu as pltpu

W = 32
M = 8192
N = 4096
CH = M // W
N2 = N // 2


def _mesh_logical_order():
    coords = [(x, y, z) for x in range(2) for y in range(4) for z in range(4)]
    ring = []
    for z in range(4):
        for yi, y in enumerate(range(4)):
            row = sorted((c for c in coords if c[1] == y and c[2] == z),
                         reverse=bool(yi % 2))
            ring.extend(row)
    return ring


def _hamiltonian_cycle():
    p0 = []
    for z in range(4):
        ys = range(4) if z % 2 == 0 else range(3, -1, -1)
        p0.extend((0, y, z) for y in ys)
    return p0 + [(1, y, z) for (_, y, z) in reversed(p0)]


_LOGICAL = _mesh_logical_order()
_L_OF = {c: i for i, c in enumerate(_LOGICAL)}
_RING = [_L_OF[c] for c in _hamiltonian_cycle()]
POS = [0] * W
SUCC = [0] * W
PRED = [0] * W
for _p, _l in enumerate(_RING):
    POS[_l] = _p
    SUCC[_l] = _RING[(_p + 1) % W]
    PRED[_l] = _RING[(_p - 1) % W]


def _ar_body(meta_ref, p_ref, o_ref,
             localR, accR, slotR0, slotR1,
             localL, accL, slotL0, slotL1,
             sendR_sem, recvR_sems, localR_sem, storeR_sems,
             sendL_sem, recvL_sems, localL_sem, storeL_sems,
             creditR, creditL):
    pos = meta_ref[0]
    succ = meta_ref[1]
    pred = meta_ref[2]
    slotR = [slotR0, slotR1]
    slotL = [slotL0, slotL1]

    barrier = pltpu.get_barrier_semaphore()
    for nbr in (pred, succ):
        pl.semaphore_signal(barrier, inc=1, device_id=(nbr,),
                            device_id_type=pl.DeviceIdType.MESH)
    pl.semaphore_wait(barrier, 2)

    def rows(c):
        return pl.ds(c * CH, CH)

    colsR = pl.ds(0, N2)
    colsL = pl.ds(N2, N2)

    cpR = pltpu.make_async_copy(p_ref.at[rows(pos), colsR], accR, localR_sem)
    cpL = pltpu.make_async_copy(p_ref.at[rows(pos), colsL], accL, localL_sem)
    cpR.start()
    cpL.start()
    cpR.wait()
    cpL.wait()

    for s in range(W - 1):
        par = s % 2
        rR = lax.rem(pos - s - 1 + 2 * W, W)
        rL = lax.rem(pos + s + 1, W)
        lpR = pltpu.make_async_copy(p_ref.at[rows(rR), colsR], localR,
                                    localR_sem)
        lpL = pltpu.make_async_copy(p_ref.at[rows(rL), colsL], localL,
                                    localL_sem)
        lpR.start()
        lpL.start()
        if s >= 2:
            pl.semaphore_wait(creditR, 1)
            pl.semaphore_wait(creditL, 1)
        rdmaR = pltpu.make_async_remote_copy(
            src_ref=accR, dst_ref=slotR[par],
            send_sem=sendR_sem, recv_sem=recvR_sems.at[par],
            device_id=(succ,), device_id_type=pl.DeviceIdType.MESH,
        )
        rdmaL = pltpu.make_async_remote_copy(
            src_ref=accL, dst_ref=slotL[par],
            send_sem=sendL_sem, recv_sem=recvL_sems.at[par],
            device_id=(pred,), device_id_type=pl.DeviceIdType.MESH,
        )
        rdmaR.start()
        rdmaL.start()
        rdmaR.wait()
        rdmaL.wait()
        lpR.wait()
        lpL.wait()
        accR[...] = slotR[par][...] + localR[...]
        accL[...] = slotL[par][...] + localL[...]
        if s <= W - 4:
            pl.semaphore_signal(creditR, inc=1, device_id=(pred,),
                                device_id_type=pl.DeviceIdType.MESH)
            pl.semaphore_signal(creditL, inc=1, device_id=(succ,),
                                device_id_type=pl.DeviceIdType.MESH)

    for _ in range(2):
        pl.semaphore_signal(creditR, inc=1, device_id=(pred,),
                            device_id_type=pl.DeviceIdType.MESH)
        pl.semaphore_signal(creditL, inc=1, device_id=(succ,),
                            device_id_type=pl.DeviceIdType.MESH)

    accR[...] = jnp.maximum(accR[...], 0.0)
    accL[...] = jnp.maximum(accL[...], 0.0)
    stR = pltpu.make_async_copy(
        accR, o_ref.at[rows(lax.rem(pos + 1, W)), colsR], storeR_sems.at[0])
    stL = pltpu.make_async_copy(
        accL, o_ref.at[rows(lax.rem(pos + W - 1, W)), colsL],
        storeL_sems.at[0])
    stR.start()
    stL.start()
    stR.wait()
    stL.wait()

    prevStR = prevStL = None
    for t in range(W - 1):
        par = t % 2
        srcR = accR if t == 0 else slotR[1 - par]
        srcL = accL if t == 0 else slotL[1 - par]
        pl.semaphore_wait(creditR, 1)
        pl.semaphore_wait(creditL, 1)
        rdmaR = pltpu.make_async_remote_copy(
            src_ref=srcR, dst_ref=slotR[par],
            send_sem=sendR_sem, recv_sem=recvR_sems.at[par],
            device_id=(succ,), device_id_type=pl.DeviceIdType.MESH,
        )
        rdmaL = pltpu.make_async_remote_copy(
            src_ref=srcL, dst_ref=slotL[par],
            send_sem=sendL_sem, recv_sem=recvL_sems.at[par],
            device_id=(pred,), device_id_type=pl.DeviceIdType.MESH,
        )
        rdmaR.start()
        rdmaL.start()
        rdmaR.wait_recv()
        rdmaL.wait_recv()
        cR = lax.rem(pos - t + 2 * W, W)
        cL = lax.rem(pos + t, W)
        stR = pltpu.make_async_copy(slotR[par], o_ref.at[rows(cR), colsR],
                                    storeR_sems.at[par])
        stL = pltpu.make_async_copy(slotL[par], o_ref.at[rows(cL), colsL],
                                    storeL_sems.at[par])
        stR.start()
        stL.start()
        rdmaR.wait_send()
        rdmaL.wait_send()
        if t >= 1:
            prevStR.wait()
            prevStL.wait()
            if t <= W - 3:
                pl.semaphore_signal(creditR, inc=1, device_id=(pred,),
                                    device_id_type=pl.DeviceIdType.MESH)
                pl.semaphore_signal(creditL, inc=1, device_id=(succ,),
                                    device_id_type=pl.DeviceIdType.MESH)
        prevStR, prevStL = stR, stL
    prevStR.wait()
    prevStL.wait()


def _allreduce_relu(meta, partial):
    return pl.pallas_call(
        _ar_body,
        out_shape=jax.ShapeDtypeStruct((M, N), jnp.float32),
        in_specs=[
            pl.BlockSpec(memory_space=pltpu.MemorySpace.SMEM),
            pl.BlockSpec(memory_space=pltpu.MemorySpace.HBM),
        ],
        out_specs=pl.BlockSpec(memory_space=pltpu.MemorySpace.HBM),
        scratch_shapes=[
            pltpu.VMEM((CH, N2), jnp.float32),
            pltpu.VMEM((CH, N2), jnp.float32),
            pltpu.VMEM((CH, N2), jnp.float32),
            pltpu.VMEM((CH, N2), jnp.float32),
            pltpu.VMEM((CH, N2), jnp.float32),
            pltpu.VMEM((CH, N2), jnp.float32),
            pltpu.VMEM((CH, N2), jnp.float32),
            pltpu.VMEM((CH, N2), jnp.float32),
            pltpu.SemaphoreType.DMA,
            pltpu.SemaphoreType.DMA((2,)),
            pltpu.SemaphoreType.DMA,
            pltpu.SemaphoreType.DMA((2,)),
            pltpu.SemaphoreType.DMA,
            pltpu.SemaphoreType.DMA((2,)),
            pltpu.SemaphoreType.DMA,
            pltpu.SemaphoreType.DMA((2,)),
            pltpu.SemaphoreType.REGULAR,
            pltpu.SemaphoreType.REGULAR,
        ],
        compiler_params=pltpu.CompilerParams(collective_id=0),
    )(meta, partial)


def kernel(x, w_mat):
    partial = lax.dot_general(
        x, w_mat, (((1,), (0,)), ((), ())),
        preferred_element_type=jnp.float32,
    )
    my = lax.axis_index("i")
    meta = jnp.stack([
        jnp.array(POS, jnp.int32)[my],
        jnp.array(SUCC, jnp.int32)[my],
        jnp.array(PRED, jnp.int32)[my],
    ])
    return _allreduce_relu(meta, partial)


# device time: 1642879 ns/iter; 2.1378x vs baseline; 1.0263x over previous
import os

import jax

jax.config.update(
    "jax_compilation_cache_dir",
    os.path.join(os.path.dirname(os.path.abspath(__file__)), "jax_cache"),
)
jax.config.update("jax_persistent_cache_min_entry_size_bytes", -1)
jax.config.update("jax_persistent_cache_min_compile_time_secs", 0.0)

import jax.numpy as jnp
from jax import lax
from jax.experimental import pallas as pl
from jax.experimental.pallas import tpu as pltpu

W = 32
M = 8192
N = 4096
CH = M // W
N2 = N // 2


def _mesh_logical_order():
    coords = [(x, y, z) for x in range(2) for y in range(4) for z in range(4)]
    ring = []
    for z in range(4):
        for yi, y in enumerate(range(4)):
            row = sorted((c for c in coords if c[1] == y and c[2] == z),
                         reverse=bool(yi % 2))
            ring.extend(row)
    return ring


def _hamiltonian_cycle():
    p0 = []
    for z in range(4):
        ys = range(4) if z % 2 == 0 else range(3, -1, -1)
        p0.extend((0, y, z) for y in ys)
    return p0 + [(1, y, z) for (_, y, z) in reversed(p0)]


_LOGICAL = _mesh_logical_order()
_L_OF = {c: i for i, c in enumerate(_LOGICAL)}
_RING = [_L_OF[c] for c in _hamiltonian_cycle()]
POS = [0] * W
SUCC = [0] * W
PRED = [0] * W
for _p, _l in enumerate(_RING):
    POS[_l] = _p
    SUCC[_l] = _RING[(_p + 1) % W]
    PRED[_l] = _RING[(_p - 1) % W]


def _ar_body(meta_ref, x_ref, w_ref, o_ref,
             localR, accR, slotR0, slotR1,
             localL, accL, slotL0, slotL1,
             sendR_sem, recvR_sems, storeR_sems,
             sendL_sem, recvL_sems, storeL_sems,
             creditR, creditL):
    pos = meta_ref[0]
    succ = meta_ref[1]
    pred = meta_ref[2]
    slotR = [slotR0, slotR1]
    slotL = [slotL0, slotL1]

    barrier = pltpu.get_barrier_semaphore()
    for nbr in (pred, succ):
        pl.semaphore_signal(barrier, inc=1, device_id=(nbr,),
                            device_id_type=pl.DeviceIdType.MESH)
    pl.semaphore_wait(barrier, 2)

    def rows(c):
        return pl.ds(c * CH, CH)

    colsR = pl.ds(0, N2)
    colsL = pl.ds(N2, N2)

    accR[...] = jnp.dot(x_ref[rows(pos), :], w_ref[:, 0:N2],
                        preferred_element_type=jnp.float32)
    accL[...] = jnp.dot(x_ref[rows(pos), :], w_ref[:, N2:N],
                        preferred_element_type=jnp.float32)

    for s in range(W - 1):
        par = s % 2
        rR = lax.rem(pos - s - 1 + 2 * W, W)
        rL = lax.rem(pos + s + 1, W)
        if s >= 2:
            pl.semaphore_wait(creditR, 1)
            pl.semaphore_wait(creditL, 1)
        rdmaR = pltpu.make_async_remote_copy(
            src_ref=accR, dst_ref=slotR[par],
            send_sem=sendR_sem, recv_sem=recvR_sems.at[par],
            device_id=(succ,), device_id_type=pl.DeviceIdType.MESH,
        )
        rdmaL = pltpu.make_async_remote_copy(
            src_ref=accL, dst_ref=slotL[par],
            send_sem=sendL_sem, recv_sem=recvL_sems.at[par],
            device_id=(pred,), device_id_type=pl.DeviceIdType.MESH,
        )
        rdmaR.start()
        rdmaL.start()
        localR[...] = jnp.dot(x_ref[rows(rR), :], w_ref[:, 0:N2],
                              preferred_element_type=jnp.float32)
        localL[...] = jnp.dot(x_ref[rows(rL), :], w_ref[:, N2:N],
                              preferred_element_type=jnp.float32)
        rdmaR.wait()
        rdmaL.wait()
        accR[...] = slotR[par][...] + localR[...]
        accL[...] = slotL[par][...] + localL[...]
        if s <= W - 4:
            pl.semaphore_signal(creditR, inc=1, device_id=(pred,),
                                device_id_type=pl.DeviceIdType.MESH)
            pl.semaphore_signal(creditL, inc=1, device_id=(succ,),
                                device_id_type=pl.DeviceIdType.MESH)

    for _ in range(2):
        pl.semaphore_signal(creditR, inc=1, device_id=(pred,),
                            device_id_type=pl.DeviceIdType.MESH)
        pl.semaphore_signal(creditL, inc=1, device_id=(succ,),
                            device_id_type=pl.DeviceIdType.MESH)

    accR[...] = jnp.maximum(accR[...], 0.0)
    accL[...] = jnp.maximum(accL[...], 0.0)
    stR = pltpu.make_async_copy(
        accR, o_ref.at[rows(lax.rem(pos + 1, W)), colsR], storeR_sems.at[0])
    stL = pltpu.make_async_copy(
        accL, o_ref.at[rows(lax.rem(pos + W - 1, W)), colsL],
        storeL_sems.at[0])
    stR.start()
    stL.start()
    stR.wait()
    stL.wait()

    prevStR = prevStL = None
    for t in range(W - 1):
        par = t % 2
        srcR = accR if t == 0 else slotR[1 - par]
        srcL = accL if t == 0 else slotL[1 - par]
        pl.semaphore_wait(creditR, 1)
        pl.semaphore_wait(creditL, 1)
        rdmaR = pltpu.make_async_remote_copy(
            src_ref=srcR, dst_ref=slotR[par],
            send_sem=sendR_sem, recv_sem=recvR_sems.at[par],
            device_id=(succ,), device_id_type=pl.DeviceIdType.MESH,
        )
        rdmaL = pltpu.make_async_remote_copy(
            src_ref=srcL, dst_ref=slotL[par],
            send_sem=sendL_sem, recv_sem=recvL_sems.at[par],
            device_id=(pred,), device_id_type=pl.DeviceIdType.MESH,
        )
        rdmaR.start()
        rdmaL.start()
        rdmaR.wait_recv()
        rdmaL.wait_recv()
        cR = lax.rem(pos - t + 2 * W, W)
        cL = lax.rem(pos + t, W)
        stR = pltpu.make_async_copy(slotR[par], o_ref.at[rows(cR), colsR],
                                    storeR_sems.at[par])
        stL = pltpu.make_async_copy(slotL[par], o_ref.at[rows(cL), colsL],
                                    storeL_sems.at[par])
        stR.start()
        stL.start()
        rdmaR.wait_send()
        rdmaL.wait_send()
        if t >= 1:
            prevStR.wait()
            prevStL.wait()
            if t <= W - 3:
                pl.semaphore_signal(creditR, inc=1, device_id=(pred,),
                                    device_id_type=pl.DeviceIdType.MESH)
                pl.semaphore_signal(creditL, inc=1, device_id=(succ,),
                                    device_id_type=pl.DeviceIdType.MESH)
        prevStR, prevStL = stR, stL
    prevStR.wait()
    prevStL.wait()


def _gemm_ar_relu(meta, x, w_mat):
    return pl.pallas_call(
        _ar_body,
        out_shape=jax.ShapeDtypeStruct((M, N), jnp.float32),
        in_specs=[
            pl.BlockSpec(memory_space=pltpu.MemorySpace.SMEM),
            pl.BlockSpec(memory_space=pltpu.MemorySpace.VMEM),
            pl.BlockSpec(memory_space=pltpu.MemorySpace.VMEM),
        ],
        out_specs=pl.BlockSpec(memory_space=pltpu.MemorySpace.HBM),
        scratch_shapes=[
            pltpu.VMEM((CH, N2), jnp.float32),
            pltpu.VMEM((CH, N2), jnp.float32),
            pltpu.VMEM((CH, N2), jnp.float32),
            pltpu.VMEM((CH, N2), jnp.float32),
            pltpu.VMEM((CH, N2), jnp.float32),
            pltpu.VMEM((CH, N2), jnp.float32),
            pltpu.VMEM((CH, N2), jnp.float32),
            pltpu.VMEM((CH, N2), jnp.float32),
            pltpu.SemaphoreType.DMA,
            pltpu.SemaphoreType.DMA((2,)),
            pltpu.SemaphoreType.DMA((2,)),
            pltpu.SemaphoreType.DMA,
            pltpu.SemaphoreType.DMA((2,)),
            pltpu.SemaphoreType.DMA((2,)),
            pltpu.SemaphoreType.REGULAR,
            pltpu.SemaphoreType.REGULAR,
        ],
        compiler_params=pltpu.CompilerParams(collective_id=0),
    )(meta, x, w_mat)


def kernel(x, w_mat):
    my = lax.axis_index("i")
    meta = jnp.stack([
        jnp.array(POS, jnp.int32)[my],
        jnp.array(SUCC, jnp.int32)[my],
        jnp.array(PRED, jnp.int32)[my],
    ])
    return _gemm_ar_relu(meta, x, w_mat)
